# Initial kernel scaffold; baseline (speedup 1.0000x reference)
import jax
import jax.numpy as jnp
from jax.experimental import pallas as pl


def kernel(ent_table, rel_table, node_ids, edge_index, edge_rel_ids):
    raise NotImplementedError("write your pallas kernel here")


# speedup vs baseline: 4.4110x; 4.4110x over previous
import functools

import jax
import jax.numpy as jnp
from jax import lax
from jax.experimental import pallas as pl
from jax.experimental.pallas import tpu as pltpu
from jax.experimental.pallas import tpu_sc as plsc

_D = 128
_N_EDGES = 320000
_N_NODES = 10000
_N_NODES_PAD = 10240
_NODES_PER_TILE = 640
_EDGES_PER_TILE = _N_EDGES // 32
_GCHUNK = 128
_ROW_BLOCK = 2000


def _rowsum_body(x_ref, o_ref):
    o_ref[...] = jnp.sum(x_ref[...], axis=1, keepdims=True)


def _rowsum(table):
    n = table.shape[0]
    out = pl.pallas_call(
        _rowsum_body,
        grid=(n // _ROW_BLOCK,),
        in_specs=[pl.BlockSpec((_ROW_BLOCK, _D), lambda i: (i, 0))],
        out_specs=pl.BlockSpec((_ROW_BLOCK, 1), lambda i: (i, 0)),
        out_shape=jax.ShapeDtypeStruct((n, 1), jnp.float32),
    )(table)
    return out.reshape(n)


def _make_sc_combine():
    mesh = plsc.VectorSubcoreMesh(core_axis_name="c", subcore_axis_name="s")

    @functools.partial(
        pl.kernel,
        out_type=jax.ShapeDtypeStruct((_N_EDGES,), jnp.float32),
        mesh=mesh,
        compiler_params=pltpu.CompilerParams(needs_layout_passes=False),
        scratch_types=[
            pltpu.VMEM((_NODES_PER_TILE,), jnp.int32),
            pltpu.VMEM((_NODES_PER_TILE,), jnp.float32),
            pltpu.VMEM_SHARED((_N_NODES_PAD,), jnp.float32),
            pltpu.VMEM((_N_NODES_PAD,), jnp.float32),
            pltpu.VMEM((_EDGES_PER_TILE,), jnp.int32),
            pltpu.VMEM((_EDGES_PER_TILE,), jnp.int32),
            pltpu.VMEM((_EDGES_PER_TILE,), jnp.int32),
            pltpu.VMEM((_EDGES_PER_TILE,), jnp.float32),
            pltpu.VMEM((_EDGES_PER_TILE,), jnp.float32),
            pltpu.SemaphoreType.DMA,
            pltpu.SemaphoreType.DMA,
        ],
    )
    def sc_combine(se_hbm, sr_hbm, nid_hbm, src_hbm, dst_hbm, rel_hbm,
                   out_hbm, nid_v, nsum_v, sh_shared, sh_v, src_v, dst_v,
                   rel_v, r_v, out_v, sem1, sem2):
        cid = lax.axis_index("c")
        sid = lax.axis_index("s")
        wid = sid * 2 + cid

        nbase = pl.multiple_of(sid * _NODES_PER_TILE, 8)
        pltpu.sync_copy(nid_hbm.at[pl.ds(nbase, _NODES_PER_TILE)], nid_v)
        ph1 = []
        for j in range(_NODES_PER_TILE // _GCHUNK):
            ph1.append(pltpu.async_copy(
                se_hbm.at[nid_v.at[pl.ds(j * _GCHUNK, _GCHUNK)]],
                nsum_v.at[pl.ds(j * _GCHUNK, _GCHUNK)], sem1))
        for h in ph1:
            h.wait()
        pltpu.sync_copy(nsum_v, sh_shared.at[pl.ds(nbase, _NODES_PER_TILE)])
        plsc.subcore_barrier()
        pltpu.sync_copy(sh_shared, sh_v)

        ebase = pl.multiple_of(wid * _EDGES_PER_TILE, 8)
        pltpu.sync_copy(src_hbm.at[pl.ds(ebase, _EDGES_PER_TILE)], src_v)
        pltpu.sync_copy(dst_hbm.at[pl.ds(ebase, _EDGES_PER_TILE)], dst_v)
        pltpu.sync_copy(rel_hbm.at[pl.ds(ebase, _EDGES_PER_TILE)], rel_v)

        handles = []
        nfull = _EDGES_PER_TILE // _GCHUNK
        for j in range(nfull):
            handles.append(pltpu.async_copy(
                sr_hbm.at[rel_v.at[pl.ds(j * _GCHUNK, _GCHUNK)]],
                r_v.at[pl.ds(j * _GCHUNK, _GCHUNK)], sem2))
            if len(handles) >= 13:
                for h in handles:
                    h.wait()
                handles = []
        rem = _EDGES_PER_TILE - nfull * _GCHUNK
        if rem:
            handles.append(pltpu.async_copy(
                sr_hbm.at[rel_v.at[pl.ds(nfull * _GCHUNK, rem)]],
                r_v.at[pl.ds(nfull * _GCHUNK, rem)], sem2))
        for h in handles:
            h.wait()

        def body(i, carry):
            o = pl.multiple_of(i * 16, 16)
            s16 = src_v[pl.ds(o, 16)]
            d16 = dst_v[pl.ds(o, 16)]
            hvec = plsc.load_gather(sh_v, [s16])
            tvec = plsc.load_gather(sh_v, [d16])
            out_v[pl.ds(o, 16)] = hvec + r_v[pl.ds(o, 16)] - tvec
            return carry

        lax.fori_loop(0, _EDGES_PER_TILE // 16, body, 0)
        pltpu.sync_copy(out_v, out_hbm.at[pl.ds(ebase, _EDGES_PER_TILE)])

    return sc_combine


_sc_combine = _make_sc_combine()


def kernel(ent_table, rel_table, node_ids, edge_index, edge_rel_ids):
    se = _rowsum(ent_table)
    sr = _rowsum(rel_table)
    nid_pad = jnp.concatenate(
        [node_ids, jnp.zeros((_N_NODES_PAD - _N_NODES,), jnp.int32)])
    src = edge_index[0]
    dst = edge_index[1]
    return _sc_combine(se, sr, nid_pad, src, dst, edge_rel_ids)


# speedup vs baseline: 5.9483x; 1.3485x over previous
import functools

import jax
import jax.numpy as jnp
from jax import lax
from jax.experimental import pallas as pl
from jax.experimental.pallas import tpu as pltpu
from jax.experimental.pallas import tpu_sc as plsc

_D = 128
_N_EDGES = 320000
_N_NODES = 10000
_N_NODES_PAD = 10240
_NODES_PER_TILE = 640
_EDGES_PER_TILE = _N_EDGES // 32
_GCHUNK = 128
_ROW_BLOCK = 10000


def _rowsum_body(x_ref, o_ref):
    ones = jnp.ones((_D, 1), jnp.float32)
    o_ref[...] = jax.lax.dot_general(
        x_ref[...], ones, (((1,), (0,)), ((), ())),
        preferred_element_type=jnp.float32)


def _rowsum_body2(x_ref, y_ref, ox_ref, oy_ref):
    ones = jnp.ones((_D, 1), jnp.float32)
    dn = (((1,), (0,)), ((), ()))
    ox_ref[...] = jax.lax.dot_general(
        x_ref[...], ones, dn, preferred_element_type=jnp.float32)
    oy_ref[...] = jax.lax.dot_general(
        y_ref[...], ones, dn, preferred_element_type=jnp.float32)


def _rowsum(table):
    n = table.shape[0]
    h = n // 2
    nblk = h // _ROW_BLOCK
    o1, o2 = pl.pallas_call(
        _rowsum_body2,
        grid=(nblk,),
        in_specs=[
            pl.BlockSpec((_ROW_BLOCK, _D), lambda i: (i, 0)),
            pl.BlockSpec((_ROW_BLOCK, _D), lambda i: (i + nblk, 0)),
        ],
        out_specs=[
            pl.BlockSpec((_ROW_BLOCK, 1), lambda i: (i, 0)),
            pl.BlockSpec((_ROW_BLOCK, 1), lambda i: (i, 0)),
        ],
        out_shape=[
            jax.ShapeDtypeStruct((h, 1), jnp.float32),
            jax.ShapeDtypeStruct((h, 1), jnp.float32),
        ],
    )(table, table)
    return jnp.concatenate([o1.reshape(h), o2.reshape(h)])


def _make_sc_combine():
    mesh = plsc.VectorSubcoreMesh(core_axis_name="c", subcore_axis_name="s")

    @functools.partial(
        pl.kernel,
        out_type=jax.ShapeDtypeStruct((_N_EDGES,), jnp.float32),
        mesh=mesh,
        compiler_params=pltpu.CompilerParams(needs_layout_passes=False),
        scratch_types=[
            pltpu.VMEM((_NODES_PER_TILE,), jnp.int32),
            pltpu.VMEM((_NODES_PER_TILE,), jnp.float32),
            pltpu.VMEM_SHARED((_N_NODES_PAD,), jnp.float32),
            pltpu.VMEM((_N_NODES_PAD,), jnp.float32),
            pltpu.VMEM((_EDGES_PER_TILE,), jnp.int32),
            pltpu.VMEM((_EDGES_PER_TILE,), jnp.int32),
            pltpu.VMEM((_EDGES_PER_TILE,), jnp.int32),
            pltpu.VMEM((_EDGES_PER_TILE,), jnp.float32),
            pltpu.VMEM((_EDGES_PER_TILE,), jnp.float32),
            pltpu.SemaphoreType.DMA,
            pltpu.SemaphoreType.DMA,
        ],
    )
    def sc_combine(se_hbm, sr_hbm, nid_hbm, src_hbm, dst_hbm, rel_hbm,
                   out_hbm, nid_v, nsum_v, sh_shared, sh_v, src_v, dst_v,
                   rel_v, r_v, out_v, sem1, sem2):
        cid = lax.axis_index("c")
        sid = lax.axis_index("s")
        wid = sid * 2 + cid

        nbase = pl.multiple_of(sid * _NODES_PER_TILE, 8)
        pltpu.sync_copy(nid_hbm.at[pl.ds(nbase, _NODES_PER_TILE)], nid_v)
        ph1 = []
        for j in range(_NODES_PER_TILE // _GCHUNK):
            ph1.append(pltpu.async_copy(
                se_hbm.at[nid_v.at[pl.ds(j * _GCHUNK, _GCHUNK)]],
                nsum_v.at[pl.ds(j * _GCHUNK, _GCHUNK)], sem1))
        for h in ph1:
            h.wait()
        pltpu.sync_copy(nsum_v, sh_shared.at[pl.ds(nbase, _NODES_PER_TILE)])
        plsc.subcore_barrier()
        pltpu.sync_copy(sh_shared, sh_v)

        ebase = pl.multiple_of(wid * _EDGES_PER_TILE, 8)
        pltpu.sync_copy(src_hbm.at[pl.ds(ebase, _EDGES_PER_TILE)], src_v)
        pltpu.sync_copy(dst_hbm.at[pl.ds(ebase, _EDGES_PER_TILE)], dst_v)
        pltpu.sync_copy(rel_hbm.at[pl.ds(ebase, _EDGES_PER_TILE)], rel_v)

        handles = []
        nfull = _EDGES_PER_TILE // _GCHUNK
        for j in range(nfull):
            handles.append(pltpu.async_copy(
                sr_hbm.at[rel_v.at[pl.ds(j * _GCHUNK, _GCHUNK)]],
                r_v.at[pl.ds(j * _GCHUNK, _GCHUNK)], sem2))
            if len(handles) >= 13:
                for h in handles:
                    h.wait()
                handles = []
        rem = _EDGES_PER_TILE - nfull * _GCHUNK
        if rem:
            handles.append(pltpu.async_copy(
                sr_hbm.at[rel_v.at[pl.ds(nfull * _GCHUNK, rem)]],
                r_v.at[pl.ds(nfull * _GCHUNK, rem)], sem2))
        for h in handles:
            h.wait()

        def body(i, carry):
            o = pl.multiple_of(i * 16, 16)
            s16 = src_v[pl.ds(o, 16)]
            d16 = dst_v[pl.ds(o, 16)]
            hvec = plsc.load_gather(sh_v, [s16])
            tvec = plsc.load_gather(sh_v, [d16])
            out_v[pl.ds(o, 16)] = hvec + r_v[pl.ds(o, 16)] - tvec
            return carry

        lax.fori_loop(0, _EDGES_PER_TILE // 16, body, 0)
        pltpu.sync_copy(out_v, out_hbm.at[pl.ds(ebase, _EDGES_PER_TILE)])

    return sc_combine


_sc_combine = _make_sc_combine()


def kernel(ent_table, rel_table, node_ids, edge_index, edge_rel_ids):
    se = _rowsum(ent_table)
    sr = _rowsum(rel_table)
    nid_pad = jnp.concatenate(
        [node_ids, jnp.zeros((_N_NODES_PAD - _N_NODES,), jnp.int32)])
    src = edge_index[0]
    dst = edge_index[1]
    return _sc_combine(se, sr, nid_pad, src, dst, edge_rel_ids)
